# baseline (speedup 1.0000x reference)
import jax
import jax.numpy as jnp
from jax import lax
from jax.experimental import pallas as pl
from jax.experimental.pallas import tpu as pltpu
from jax.experimental.pallas import tpu_sc as plsc

_NC, _NS = 2, 16
_NW = _NC * _NS
_CHUNK = 64


def _sc_body(table_hbm, out_hbm, buf, sem):
    c = lax.axis_index("c")
    s = lax.axis_index("s")
    wid = s * _NC + c
    rows_per_w = table_hbm.shape[0] // _NW
    base = wid * rows_per_w
    nb = out_hbm.shape[0]
    for k in range(rows_per_w // _CHUNK):
        r0 = base + k * _CHUNK
        pltpu.sync_copy(table_hbm.at[pl.ds(r0, _CHUNK), :], buf)
        for b in range(nb):
            pltpu.async_copy(buf, out_hbm.at[b, pl.ds(r0, _CHUNK), :], sem)
        for b in range(nb):
            pltpu.make_async_copy(buf, out_hbm.at[0, pl.ds(r0, _CHUNK), :], sem).wait()


def _sc_copy(table, n_batch):
    seq_len, embed_dim = table.shape
    mesh = plsc.VectorSubcoreMesh(core_axis_name="c", subcore_axis_name="s")
    fn = pl.kernel(
        _sc_body,
        out_type=jax.ShapeDtypeStruct((n_batch, seq_len, embed_dim), table.dtype),
        mesh=mesh,
        scratch_types=[
            pltpu.VMEM((_CHUNK, embed_dim), table.dtype),
            pltpu.SemaphoreType.DMA,
        ],
    )
    return fn(table)


def _bcast_body(w_ref, o_ref):
    o_ref[...] = jnp.broadcast_to(w_ref[...][None, :, :], o_ref.shape)


def _tc_copy(table, n_batch):
    seq_len, embed_dim = table.shape
    block = 1024
    return pl.pallas_call(
        _bcast_body,
        grid=(seq_len // block,),
        in_specs=[pl.BlockSpec((block, embed_dim), lambda i: (i, 0))],
        out_specs=pl.BlockSpec((n_batch, block, embed_dim), lambda i: (0, i, 0)),
        out_shape=jax.ShapeDtypeStruct((n_batch, seq_len, embed_dim), table.dtype),
    )(table)


def kernel(input_ids, embed_weight):
    batch, seq_len = input_ids.shape
    table = embed_weight[:seq_len]
    n_sc = 1
    tc_out = _tc_copy(table, batch - n_sc)
    sc_out = _sc_copy(table, n_sc)
    return jnp.concatenate([tc_out, sc_out], axis=0)


# speedup vs baseline: 3.1835x; 3.1835x over previous
import jax
import jax.numpy as jnp
from jax.experimental import pallas as pl
from jax.experimental.pallas import tpu as pltpu

_CHUNK = 512


_NBUF = 4
_LAG = 2


def _copy_body(w_hbm, o_hbm, *refs):
    bufs = refs[:_NBUF]
    in_sems = refs[_NBUF : 2 * _NBUF]
    out_sems = refs[2 * _NBUF : 3 * _NBUF]
    n = w_hbm.shape[0] // _CHUNK
    nb = o_hbm.shape[0]

    def in_cp(k):
        return pltpu.make_async_copy(
            w_hbm.at[pl.ds(k * _CHUNK, _CHUNK), :], bufs[k % _NBUF], in_sems[k % _NBUF]
        )

    def out_cp(k, b):
        return pltpu.make_async_copy(
            bufs[k % _NBUF],
            o_hbm.at[b, pl.ds(k * _CHUNK, _CHUNK), :],
            out_sems[k % _NBUF],
        )

    for j in range(min(_NBUF - _LAG, n)):
        in_cp(j).start()
    for k in range(n):
        if k >= _LAG:
            for b in range(nb):
                out_cp(k - _LAG, b).wait()
        if k + _NBUF - _LAG < n:
            in_cp(k + _NBUF - _LAG).start()
        in_cp(k).wait()
        for b in range(nb):
            out_cp(k, b).start()
    for k in range(max(0, n - _LAG), n):
        for b in range(nb):
            out_cp(k, b).wait()


def kernel(input_ids, embed_weight):
    batch, seq_len = input_ids.shape
    _, embed_dim = embed_weight.shape
    table = embed_weight[:seq_len]
    out = pl.pallas_call(
        _copy_body,
        in_specs=[pl.BlockSpec(memory_space=pl.ANY)],
        out_specs=pl.BlockSpec(memory_space=pl.ANY),
        out_shape=jax.ShapeDtypeStruct((batch, seq_len, embed_dim), table.dtype),
        scratch_shapes=(
            [pltpu.VMEM((_CHUNK, embed_dim), table.dtype) for _ in range(_NBUF)]
            + [pltpu.SemaphoreType.DMA for _ in range(2 * _NBUF)]
        ),
    )(table)
    return out


# speedup vs baseline: 3.2890x; 1.0331x over previous
import jax
import jax.numpy as jnp
from jax.experimental import pallas as pl
from jax.experimental.pallas import tpu as pltpu

_CHUNK = 1024


_NBUF = 4
_LAG = 2


def _copy_body(w_hbm, o_hbm, *refs):
    bufs = refs[:_NBUF]
    in_sems = refs[_NBUF : 2 * _NBUF]
    out_sems = refs[2 * _NBUF : 3 * _NBUF]
    n = w_hbm.shape[0] // _CHUNK
    nb = o_hbm.shape[0]

    def in_cp(k):
        return pltpu.make_async_copy(
            w_hbm.at[pl.ds(k * _CHUNK, _CHUNK), :], bufs[k % _NBUF], in_sems[k % _NBUF]
        )

    def out_cp(k, b):
        return pltpu.make_async_copy(
            bufs[k % _NBUF],
            o_hbm.at[b, pl.ds(k * _CHUNK, _CHUNK), :],
            out_sems[k % _NBUF],
        )

    for j in range(min(_NBUF - _LAG, n)):
        in_cp(j).start()
    for k in range(n):
        if k >= _LAG:
            for b in range(nb):
                out_cp(k - _LAG, b).wait()
        if k + _NBUF - _LAG < n:
            in_cp(k + _NBUF - _LAG).start()
        in_cp(k).wait()
        for b in range(nb):
            out_cp(k, b).start()
    for k in range(max(0, n - _LAG), n):
        for b in range(nb):
            out_cp(k, b).wait()


def kernel(input_ids, embed_weight):
    batch, seq_len = input_ids.shape
    _, embed_dim = embed_weight.shape
    table = embed_weight[:seq_len]
    out = pl.pallas_call(
        _copy_body,
        in_specs=[pl.BlockSpec(memory_space=pl.ANY)],
        out_specs=pl.BlockSpec(memory_space=pl.ANY),
        out_shape=jax.ShapeDtypeStruct((batch, seq_len, embed_dim), table.dtype),
        scratch_shapes=(
            [pltpu.VMEM((_CHUNK, embed_dim), table.dtype) for _ in range(_NBUF)]
            + [pltpu.SemaphoreType.DMA for _ in range(2 * _NBUF)]
        ),
    )(table)
    return out


# speedup vs baseline: 3.3225x; 1.0102x over previous
import jax
import jax.numpy as jnp
from jax.experimental import pallas as pl
from jax.experimental.pallas import tpu as pltpu

_CHUNK = 2048


_NBUF = 4
_LAG = 2


def _copy_body(w_hbm, o_hbm, *refs):
    bufs = refs[:_NBUF]
    in_sems = refs[_NBUF : 2 * _NBUF]
    out_sems = refs[2 * _NBUF : 3 * _NBUF]
    n = w_hbm.shape[0] // _CHUNK
    nb = o_hbm.shape[0]

    def in_cp(k):
        return pltpu.make_async_copy(
            w_hbm.at[pl.ds(k * _CHUNK, _CHUNK), :], bufs[k % _NBUF], in_sems[k % _NBUF]
        )

    def out_cp(k, b):
        return pltpu.make_async_copy(
            bufs[k % _NBUF],
            o_hbm.at[b, pl.ds(k * _CHUNK, _CHUNK), :],
            out_sems[k % _NBUF],
        )

    for j in range(min(_NBUF - _LAG, n)):
        in_cp(j).start()
    for k in range(n):
        if k >= _LAG:
            for b in range(nb):
                out_cp(k - _LAG, b).wait()
        if k + _NBUF - _LAG < n:
            in_cp(k + _NBUF - _LAG).start()
        in_cp(k).wait()
        for b in range(nb):
            out_cp(k, b).start()
    for k in range(max(0, n - _LAG), n):
        for b in range(nb):
            out_cp(k, b).wait()


def kernel(input_ids, embed_weight):
    batch, seq_len = input_ids.shape
    _, embed_dim = embed_weight.shape
    table = embed_weight[:seq_len]
    out = pl.pallas_call(
        _copy_body,
        in_specs=[pl.BlockSpec(memory_space=pl.ANY)],
        out_specs=pl.BlockSpec(memory_space=pl.ANY),
        out_shape=jax.ShapeDtypeStruct((batch, seq_len, embed_dim), table.dtype),
        scratch_shapes=(
            [pltpu.VMEM((_CHUNK, embed_dim), table.dtype) for _ in range(_NBUF)]
            + [pltpu.SemaphoreType.DMA for _ in range(2 * _NBUF)]
        ),
    )(table)
    return out
